# baseline (speedup 1.0000x reference)
import jax
import jax.numpy as jnp
from jax.experimental import pallas as pl
from jax.experimental.pallas import tpu as pltpu

_BETA = 0.25
_NROWS = 9216
_D = 256
_K = 1024
_BLK = 1536
_NBLK = _NROWS // _BLK
_NLEV = 4


def _rvq_kernel(x_ref, e0_ref, e1_ref, e2_ref, e3_ref,
                fq_ref, q0_ref, q1_ref, q2_ref, q3_ref,
                i0_ref, i1_ref, i2_ref, i3_ref,
                loss_ref, perp_ref,
                counts_ref, sse_ref, esplit_ref, esumt_ref):
    i = pl.program_id(0)
    e_refs = (e0_ref, e1_ref, e2_ref, e3_ref)
    q_refs = (q0_ref, q1_ref, q2_ref, q3_ref)
    i_refs = (i0_ref, i1_ref, i2_ref, i3_ref)

    @pl.when(i == 0)
    def _init():
        counts_ref[...] = jnp.zeros_like(counts_ref)
        sse_ref[...] = jnp.zeros_like(sse_ref)
        for k in range(_NLEV):
            e = e_refs[k][...]
            hi = e.astype(jnp.bfloat16)
            r1 = e - hi.astype(jnp.float32)
            mid = r1.astype(jnp.bfloat16)
            r2 = r1 - mid.astype(jnp.float32)
            esplit_ref[3 * k + 0] = hi
            esplit_ref[3 * k + 1] = mid
            esplit_ref[3 * k + 2] = r2.astype(jnp.bfloat16)
            esum = jnp.sum(e * e, axis=1, keepdims=True)
            esumt_ref[k:k + 1, :] = esum.T

    _NCH = 3
    _H = _BLK // _NCH
    lane = jax.lax.broadcasted_iota(
        jnp.int32, (_H, _K), 1).astype(jnp.float32)
    rs = [x_ref[pl.ds(s * _H, _H), :] for s in range(_NCH)]
    fqs = [None] * _NCH
    dn = (((1,), (0,)), ((), ()))
    for k in range(_NLEV):
        emb = e_refs[k][...]
        sse_parts = []
        cnt_parts = []
        for s in range(_NCH):
            r = rs[s]
            rsum = jnp.sum(r * r, axis=1, keepdims=True)
            dot = jax.lax.dot_general(
                r, emb, (((1,), (1,)), ((), ())),
                preferred_element_type=jnp.float32)
            dist = rsum + esumt_ref[k:k + 1, :] - 2.0 * dot
            md = jnp.min(dist, axis=1, keepdims=True)
            idxf = jnp.min(
                jnp.where(dist == md, lane, jnp.float32(_K)), axis=1)
            idx = idxf.astype(jnp.int32)
            hot = lane == idxf[:, None]
            onehot = hot.astype(jnp.float32)
            oh_b = hot.astype(jnp.bfloat16)
            z_q = (jax.lax.dot_general(oh_b, esplit_ref[3 * k + 0], dn,
                                       preferred_element_type=jnp.float32)
                   + jax.lax.dot_general(oh_b, esplit_ref[3 * k + 1], dn,
                                         preferred_element_type=jnp.float32)
                   + jax.lax.dot_general(oh_b, esplit_ref[3 * k + 2], dn,
                                         preferred_element_type=jnp.float32))
            t = z_q - r
            q_st = r + t
            q_refs[k][pl.ds(s * _H, _H), :] = q_st
            i_refs[k][0, 0, pl.ds(s * _H, _H)] = idx
            sse_parts.append(jnp.sum(t * t, axis=0, keepdims=True))
            cnt_parts.append(jnp.sum(onehot, axis=0, keepdims=True))
            fqs[s] = q_st if k == 0 else fqs[s] + q_st
            rs[s] = r - q_st
        sse_ref[k:k + 1, :] += sum(sse_parts[1:], sse_parts[0])
        counts_ref[k:k + 1, :] += sum(cnt_parts[1:], cnt_parts[0])
    for s in range(_NCH):
        fq_ref[pl.ds(s * _H, _H), :] = fqs[s]

    @pl.when(i == _NBLK - 1)
    def _fin():
        n = jnp.float32(_NROWS * _D)
        total = jnp.float32(0.0)
        for k in range(_NLEV):
            m = jnp.sum(sse_ref[k, :]) / n
            total = total + (_BETA * m + m)
        loss_ref[...] = jnp.full((8, 128), total, jnp.float32)
        probs = counts_ref[...] / jnp.float32(_NROWS)
        ent = jnp.sum(probs * jnp.log(probs + 1e-10), axis=1)
        perp = jnp.exp(-ent)
        perp_ref[...] = jnp.broadcast_to(perp[:, None], (8, 128))


def kernel(x, e0, e1, e2, e3):
    zf = x.reshape(_NROWS, _D)
    f32 = jnp.float32
    out_shapes = (
        jax.ShapeDtypeStruct((_NROWS, _D), f32),
        jax.ShapeDtypeStruct((_NROWS, _D), f32),
        jax.ShapeDtypeStruct((_NROWS, _D), f32),
        jax.ShapeDtypeStruct((_NROWS, _D), f32),
        jax.ShapeDtypeStruct((_NROWS, _D), f32),
        jax.ShapeDtypeStruct((_NBLK, 1, _BLK), jnp.int32),
        jax.ShapeDtypeStruct((_NBLK, 1, _BLK), jnp.int32),
        jax.ShapeDtypeStruct((_NBLK, 1, _BLK), jnp.int32),
        jax.ShapeDtypeStruct((_NBLK, 1, _BLK), jnp.int32),
        jax.ShapeDtypeStruct((8, 128), f32),
        jax.ShapeDtypeStruct((8, 128), f32),
    )
    row_spec = pl.BlockSpec((_BLK, _D), lambda i: (i, 0))
    emb_spec = pl.BlockSpec((_K, _D), lambda i: (0, 0))
    idx_spec = pl.BlockSpec((1, 1, _BLK), lambda i: (i, 0, 0))
    scl_spec = pl.BlockSpec((8, 128), lambda i: (0, 0))
    outs = pl.pallas_call(
        _rvq_kernel,
        grid=(_NBLK,),
        in_specs=[row_spec, emb_spec, emb_spec, emb_spec, emb_spec],
        out_specs=(row_spec, row_spec, row_spec, row_spec, row_spec,
                   idx_spec, idx_spec, idx_spec, idx_spec,
                   scl_spec, scl_spec),
        out_shape=out_shapes,
        scratch_shapes=[pltpu.VMEM((8, _K), f32),
                        pltpu.VMEM((8, _D), f32),
                        pltpu.VMEM((3 * _NLEV, _K, _D), jnp.bfloat16),
                        pltpu.VMEM((8, _K), f32)],
        compiler_params=pltpu.CompilerParams(
            dimension_semantics=("arbitrary",)),
    )(zf, e0, e1, e2, e3)
    fq, q0, q1, q2, q3, i0, i1, i2, i3, loss, perp = outs
    shp = x.shape
    ishp = x.shape[:-1]
    qs = tuple(q.reshape(shp) for q in (q0, q1, q2, q3))
    idxs = tuple(ii.reshape(ishp) for ii in (i0, i1, i2, i3))
    perps = tuple(perp[k, 0] for k in range(_NLEV))
    return (fq.reshape(shp), loss[0, 0], (perps, qs, idxs))


# speedup vs baseline: 1.0027x; 1.0027x over previous
import jax
import jax.numpy as jnp
from jax.experimental import pallas as pl
from jax.experimental.pallas import tpu as pltpu

_BETA = 0.25
_NROWS = 9216
_D = 256
_K = 1024
_BLK = 1536
_NBLK = _NROWS // _BLK
_NLEV = 4


def _rvq_kernel(x_ref, e0_ref, e1_ref, e2_ref, e3_ref,
                fq_ref, q0_ref, q1_ref, q2_ref, q3_ref,
                i0_ref, i1_ref, i2_ref, i3_ref,
                loss_ref, perp_ref,
                counts_ref, sse_ref, esplit_ref, esumt_ref):
    i = pl.program_id(0)
    e_refs = (e0_ref, e1_ref, e2_ref, e3_ref)
    q_refs = (q0_ref, q1_ref, q2_ref, q3_ref)
    i_refs = (i0_ref, i1_ref, i2_ref, i3_ref)

    @pl.when(i == 0)
    def _init():
        counts_ref[...] = jnp.zeros_like(counts_ref)
        sse_ref[...] = jnp.zeros_like(sse_ref)
        for k in range(_NLEV):
            e = e_refs[k][...]
            hi = e.astype(jnp.bfloat16)
            r1 = e - hi.astype(jnp.float32)
            mid = r1.astype(jnp.bfloat16)
            r2 = r1 - mid.astype(jnp.float32)
            esplit_ref[3 * k + 0] = hi
            esplit_ref[3 * k + 1] = mid
            esplit_ref[3 * k + 2] = r2.astype(jnp.bfloat16)
            esum = jnp.sum(e * e, axis=1, keepdims=True)
            esumt_ref[k:k + 1, :] = esum.T

    _NCH = 3
    _H = _BLK // _NCH
    lane = jax.lax.broadcasted_iota(
        jnp.int32, (_H, _K), 1).astype(jnp.float32)
    rs = [x_ref[pl.ds(s * _H, _H), :] for s in range(_NCH)]
    fqs = [None] * _NCH
    dn = (((1,), (0,)), ((), ()))
    ones_h = jnp.ones((8, _H), jnp.bfloat16)
    dc = (((1,), (0,)), ((), ()))
    for k in range(_NLEV):
        emb = e_refs[k][...]
        sse_parts = []
        cnt_parts = []
        for s in range(_NCH):
            r = rs[s]
            rsum = jnp.sum(r * r, axis=1, keepdims=True)
            dot = jax.lax.dot_general(
                r, emb, (((1,), (1,)), ((), ())),
                preferred_element_type=jnp.float32)
            dist = rsum + esumt_ref[k:k + 1, :] - 2.0 * dot
            md = jnp.min(dist, axis=1, keepdims=True)
            idxf = jnp.min(
                jnp.where(dist == md, lane, jnp.float32(_K)), axis=1)
            idx = idxf.astype(jnp.int32)
            hot = lane == idxf[:, None]
            oh_b = hot.astype(jnp.bfloat16)
            z_q = (jax.lax.dot_general(oh_b, esplit_ref[3 * k + 0], dn,
                                       preferred_element_type=jnp.float32)
                   + jax.lax.dot_general(oh_b, esplit_ref[3 * k + 1], dn,
                                         preferred_element_type=jnp.float32)
                   + jax.lax.dot_general(oh_b, esplit_ref[3 * k + 2], dn,
                                         preferred_element_type=jnp.float32))
            t = z_q - r
            q_st = r + t
            q_refs[k][pl.ds(s * _H, _H), :] = q_st
            i_refs[k][0, 0, pl.ds(s * _H, _H)] = idx
            sse_parts.append(jnp.sum(t * t, axis=0, keepdims=True))
            cnt_parts.append(jax.lax.dot_general(
                ones_h, oh_b, dc,
                preferred_element_type=jnp.float32)[0:1])
            fqs[s] = q_st if k == 0 else fqs[s] + q_st
            rs[s] = r - q_st
        sse_ref[k:k + 1, :] += sum(sse_parts[1:], sse_parts[0])
        counts_ref[k:k + 1, :] += sum(cnt_parts[1:], cnt_parts[0])
    for s in range(_NCH):
        fq_ref[pl.ds(s * _H, _H), :] = fqs[s]

    @pl.when(i == _NBLK - 1)
    def _fin():
        n = jnp.float32(_NROWS * _D)
        total = jnp.float32(0.0)
        for k in range(_NLEV):
            m = jnp.sum(sse_ref[k, :]) / n
            total = total + (_BETA * m + m)
        loss_ref[...] = jnp.full((8, 128), total, jnp.float32)
        probs = counts_ref[...] / jnp.float32(_NROWS)
        ent = jnp.sum(probs * jnp.log(probs + 1e-10), axis=1)
        perp = jnp.exp(-ent)
        perp_ref[...] = jnp.broadcast_to(perp[:, None], (8, 128))


def kernel(x, e0, e1, e2, e3):
    zf = x.reshape(_NROWS, _D)
    f32 = jnp.float32
    out_shapes = (
        jax.ShapeDtypeStruct((_NROWS, _D), f32),
        jax.ShapeDtypeStruct((_NROWS, _D), f32),
        jax.ShapeDtypeStruct((_NROWS, _D), f32),
        jax.ShapeDtypeStruct((_NROWS, _D), f32),
        jax.ShapeDtypeStruct((_NROWS, _D), f32),
        jax.ShapeDtypeStruct((_NBLK, 1, _BLK), jnp.int32),
        jax.ShapeDtypeStruct((_NBLK, 1, _BLK), jnp.int32),
        jax.ShapeDtypeStruct((_NBLK, 1, _BLK), jnp.int32),
        jax.ShapeDtypeStruct((_NBLK, 1, _BLK), jnp.int32),
        jax.ShapeDtypeStruct((8, 128), f32),
        jax.ShapeDtypeStruct((8, 128), f32),
    )
    row_spec = pl.BlockSpec((_BLK, _D), lambda i: (i, 0))
    emb_spec = pl.BlockSpec((_K, _D), lambda i: (0, 0))
    idx_spec = pl.BlockSpec((1, 1, _BLK), lambda i: (i, 0, 0))
    scl_spec = pl.BlockSpec((8, 128), lambda i: (0, 0))
    outs = pl.pallas_call(
        _rvq_kernel,
        grid=(_NBLK,),
        in_specs=[row_spec, emb_spec, emb_spec, emb_spec, emb_spec],
        out_specs=(row_spec, row_spec, row_spec, row_spec, row_spec,
                   idx_spec, idx_spec, idx_spec, idx_spec,
                   scl_spec, scl_spec),
        out_shape=out_shapes,
        scratch_shapes=[pltpu.VMEM((8, _K), f32),
                        pltpu.VMEM((8, _D), f32),
                        pltpu.VMEM((3 * _NLEV, _K, _D), jnp.bfloat16),
                        pltpu.VMEM((8, _K), f32)],
        compiler_params=pltpu.CompilerParams(
            dimension_semantics=("arbitrary",)),
    )(zf, e0, e1, e2, e3)
    fq, q0, q1, q2, q3, i0, i1, i2, i3, loss, perp = outs
    shp = x.shape
    ishp = x.shape[:-1]
    qs = tuple(q.reshape(shp) for q in (q0, q1, q2, q3))
    idxs = tuple(ii.reshape(ishp) for ii in (i0, i1, i2, i3))
    perps = tuple(perp[k, 0] for k in range(_NLEV))
    return (fq.reshape(shp), loss[0, 0], (perps, qs, idxs))


# speedup vs baseline: 1.0043x; 1.0016x over previous
import jax
import jax.numpy as jnp
from jax.experimental import pallas as pl
from jax.experimental.pallas import tpu as pltpu

_BETA = 0.25
_NROWS = 9216
_D = 256
_K = 1024
_BLK = 1536
_NBLK = _NROWS // _BLK
_NLEV = 4


def _rvq_kernel(x_ref, e0_ref, e1_ref, e2_ref, e3_ref,
                fq_ref, q0_ref, q1_ref, q2_ref, q3_ref,
                i0_ref, i1_ref, i2_ref, i3_ref,
                loss_ref, perp_ref,
                counts_ref, sse_ref, esplit_ref, esumt_ref):
    i = pl.program_id(0)
    e_refs = (e0_ref, e1_ref, e2_ref, e3_ref)
    q_refs = (q0_ref, q1_ref, q2_ref, q3_ref)
    i_refs = (i0_ref, i1_ref, i2_ref, i3_ref)

    @pl.when(i == 0)
    def _init():
        counts_ref[...] = jnp.zeros_like(counts_ref)
        sse_ref[...] = jnp.zeros_like(sse_ref)
        for k in range(_NLEV):
            e = e_refs[k][...]
            hi = e.astype(jnp.bfloat16)
            r1 = e - hi.astype(jnp.float32)
            mid = r1.astype(jnp.bfloat16)
            r2 = r1 - mid.astype(jnp.float32)
            esplit_ref[k, :, 0:_D] = hi
            esplit_ref[k, :, _D:2 * _D] = mid
            esplit_ref[k, :, 2 * _D:3 * _D] = r2.astype(jnp.bfloat16)
            esum = jnp.sum(e * e, axis=1, keepdims=True)
            esumt_ref[k:k + 1, :] = esum.T

    _NCH = 3
    _H = _BLK // _NCH
    lane = jax.lax.broadcasted_iota(
        jnp.int32, (_H, _K), 1).astype(jnp.float32)
    rs = [x_ref[pl.ds(s * _H, _H), :] for s in range(_NCH)]
    fqs = [None] * _NCH
    dn = (((1,), (0,)), ((), ()))
    ones_h = jnp.ones((8, _H), jnp.bfloat16)
    dc = (((1,), (0,)), ((), ()))
    for k in range(_NLEV):
        emb = e_refs[k][...]
        sse_parts = []
        cnt_parts = []
        for s in range(_NCH):
            r = rs[s]
            rsum = jnp.sum(r * r, axis=1, keepdims=True)
            dot = jax.lax.dot_general(
                r, emb, (((1,), (1,)), ((), ())),
                preferred_element_type=jnp.float32)
            dist = rsum + esumt_ref[k:k + 1, :] - 2.0 * dot
            md = jnp.min(dist, axis=1, keepdims=True)
            idxf = jnp.min(
                jnp.where(dist == md, lane, jnp.float32(_K)), axis=1)
            idx = idxf.astype(jnp.int32)
            hot = lane == idxf[:, None]
            oh_b = hot.astype(jnp.bfloat16)
            g = jax.lax.dot_general(oh_b, esplit_ref[k], dn,
                                    preferred_element_type=jnp.float32)
            z_q = ((g[:, 0:_D] + g[:, _D:2 * _D]) + g[:, 2 * _D:3 * _D])
            t = z_q - r
            q_st = r + t
            q_refs[k][pl.ds(s * _H, _H), :] = q_st
            i_refs[k][0, 0, pl.ds(s * _H, _H)] = idx
            sse_parts.append(jnp.sum(t * t, axis=0, keepdims=True))
            cnt_parts.append(jax.lax.dot_general(
                ones_h, oh_b, dc,
                preferred_element_type=jnp.float32)[0:1])
            fqs[s] = q_st if k == 0 else fqs[s] + q_st
            rs[s] = r - q_st
        sse_ref[k:k + 1, :] += sum(sse_parts[1:], sse_parts[0])
        counts_ref[k:k + 1, :] += sum(cnt_parts[1:], cnt_parts[0])
    for s in range(_NCH):
        fq_ref[pl.ds(s * _H, _H), :] = fqs[s]

    @pl.when(i == _NBLK - 1)
    def _fin():
        n = jnp.float32(_NROWS * _D)
        total = jnp.float32(0.0)
        for k in range(_NLEV):
            m = jnp.sum(sse_ref[k, :]) / n
            total = total + (_BETA * m + m)
        loss_ref[...] = jnp.full((8, 128), total, jnp.float32)
        probs = counts_ref[...] / jnp.float32(_NROWS)
        ent = jnp.sum(probs * jnp.log(probs + 1e-10), axis=1)
        perp = jnp.exp(-ent)
        perp_ref[...] = jnp.broadcast_to(perp[:, None], (8, 128))


def kernel(x, e0, e1, e2, e3):
    zf = x.reshape(_NROWS, _D)
    f32 = jnp.float32
    out_shapes = (
        jax.ShapeDtypeStruct((_NROWS, _D), f32),
        jax.ShapeDtypeStruct((_NROWS, _D), f32),
        jax.ShapeDtypeStruct((_NROWS, _D), f32),
        jax.ShapeDtypeStruct((_NROWS, _D), f32),
        jax.ShapeDtypeStruct((_NROWS, _D), f32),
        jax.ShapeDtypeStruct((_NBLK, 1, _BLK), jnp.int32),
        jax.ShapeDtypeStruct((_NBLK, 1, _BLK), jnp.int32),
        jax.ShapeDtypeStruct((_NBLK, 1, _BLK), jnp.int32),
        jax.ShapeDtypeStruct((_NBLK, 1, _BLK), jnp.int32),
        jax.ShapeDtypeStruct((8, 128), f32),
        jax.ShapeDtypeStruct((8, 128), f32),
    )
    row_spec = pl.BlockSpec((_BLK, _D), lambda i: (i, 0))
    emb_spec = pl.BlockSpec((_K, _D), lambda i: (0, 0))
    idx_spec = pl.BlockSpec((1, 1, _BLK), lambda i: (i, 0, 0))
    scl_spec = pl.BlockSpec((8, 128), lambda i: (0, 0))
    outs = pl.pallas_call(
        _rvq_kernel,
        grid=(_NBLK,),
        in_specs=[row_spec, emb_spec, emb_spec, emb_spec, emb_spec],
        out_specs=(row_spec, row_spec, row_spec, row_spec, row_spec,
                   idx_spec, idx_spec, idx_spec, idx_spec,
                   scl_spec, scl_spec),
        out_shape=out_shapes,
        scratch_shapes=[pltpu.VMEM((8, _K), f32),
                        pltpu.VMEM((8, _D), f32),
                        pltpu.VMEM((_NLEV, _K, 3 * _D), jnp.bfloat16),
                        pltpu.VMEM((8, _K), f32)],
        compiler_params=pltpu.CompilerParams(
            dimension_semantics=("arbitrary",)),
    )(zf, e0, e1, e2, e3)
    fq, q0, q1, q2, q3, i0, i1, i2, i3, loss, perp = outs
    shp = x.shape
    ishp = x.shape[:-1]
    qs = tuple(q.reshape(shp) for q in (q0, q1, q2, q3))
    idxs = tuple(ii.reshape(ishp) for ii in (i0, i1, i2, i3))
    perps = tuple(perp[k, 0] for k in range(_NLEV))
    return (fq.reshape(shp), loss[0, 0], (perps, qs, idxs))


# speedup vs baseline: 1.0904x; 1.0857x over previous
import jax
import jax.numpy as jnp
from jax.experimental import pallas as pl
from jax.experimental.pallas import tpu as pltpu

_BETA = 0.25
_NROWS = 9216
_D = 256
_K = 1024
_BLK = 1536
_NBLK = _NROWS // _BLK
_NLEV = 4


def _rvq_kernel(x_ref, e0_ref, e1_ref, e2_ref, e3_ref,
                fq_ref, q0_ref, q1_ref, q2_ref, q3_ref,
                i0_ref, i1_ref, i2_ref, i3_ref,
                loss_ref, perp_ref,
                counts_ref, sse_ref, esplit_ref, esumt_ref):
    i = pl.program_id(0)
    e_refs = (e0_ref, e1_ref, e2_ref, e3_ref)
    q_refs = (q0_ref, q1_ref, q2_ref, q3_ref)
    i_refs = (i0_ref, i1_ref, i2_ref, i3_ref)

    @pl.when(i == 0)
    def _init():
        counts_ref[...] = jnp.zeros_like(counts_ref)
        sse_ref[...] = jnp.zeros_like(sse_ref)
        for k in range(_NLEV):
            e = e_refs[k][...]
            hi = e.astype(jnp.bfloat16)
            r1 = e - hi.astype(jnp.float32)
            mid = r1.astype(jnp.bfloat16)
            r2 = r1 - mid.astype(jnp.float32)
            esplit_ref[k, :, 0:_D] = hi
            esplit_ref[k, :, _D:2 * _D] = mid
            esplit_ref[k, :, 2 * _D:3 * _D] = r2.astype(jnp.bfloat16)
            esum = jnp.sum(e * e, axis=1, keepdims=True)
            esumt_ref[k:k + 1, :] = esum.T

    _NCH = 3
    _H = _BLK // _NCH
    lane = jax.lax.broadcasted_iota(
        jnp.int32, (_H, _K), 1).astype(jnp.float32)
    lane128 = jax.lax.broadcasted_iota(
        jnp.int32, (_H, 128), 1).astype(jnp.float32)
    rs = [x_ref[pl.ds(s * _H, _H), :] for s in range(_NCH)]
    fqs = [None] * _NCH
    dn = (((1,), (0,)), ((), ()))
    ones_h = jnp.ones((8, _H), jnp.bfloat16)
    dc = (((1,), (0,)), ((), ()))
    for k in range(_NLEV):
        emb = e_refs[k][...]
        sse_parts = []
        cnt_parts = []
        for s in range(_NCH):
            r = rs[s]
            rsum = jnp.sum(r * r, axis=1, keepdims=True)
            dot = jax.lax.dot_general(
                r, emb, (((1,), (1,)), ((), ())),
                preferred_element_type=jnp.float32)
            esr = esumt_ref[k:k + 1, :]
            m = rsum + esr[:, 0:128] - 2.0 * dot[:, 0:128]
            jv = jnp.zeros((_H, 128), jnp.float32)
            for j in range(1, _K // 128):
                c = (rsum + esr[:, 128 * j:128 * (j + 1)]
                     - 2.0 * dot[:, 128 * j:128 * (j + 1)])
                lt = c < m
                m = jnp.where(lt, c, m)
                jv = jnp.where(lt, jnp.float32(j), jv)
            mmin = jnp.min(m, axis=1, keepdims=True)
            gidx = jv * 128.0 + lane128
            idxf = jnp.min(
                jnp.where(m == mmin, gidx, jnp.float32(8192.0)), axis=1)
            idx = idxf.astype(jnp.int32)
            hot = lane == idxf[:, None]
            oh_b = hot.astype(jnp.bfloat16)
            g = jax.lax.dot_general(oh_b, esplit_ref[k], dn,
                                    preferred_element_type=jnp.float32)
            z_q = ((g[:, 0:_D] + g[:, _D:2 * _D]) + g[:, 2 * _D:3 * _D])
            t = z_q - r
            q_st = r + t
            q_refs[k][pl.ds(s * _H, _H), :] = q_st
            i_refs[k][0, 0, pl.ds(s * _H, _H)] = idx
            sse_parts.append(jnp.sum(t * t, axis=0, keepdims=True))
            cnt_parts.append(jax.lax.dot_general(
                ones_h, oh_b, dc,
                preferred_element_type=jnp.float32)[0:1])
            fqs[s] = q_st if k == 0 else fqs[s] + q_st
            rs[s] = r - q_st
        sse_ref[k:k + 1, :] += sum(sse_parts[1:], sse_parts[0])
        counts_ref[k:k + 1, :] += sum(cnt_parts[1:], cnt_parts[0])
    for s in range(_NCH):
        fq_ref[pl.ds(s * _H, _H), :] = fqs[s]

    @pl.when(i == _NBLK - 1)
    def _fin():
        n = jnp.float32(_NROWS * _D)
        total = jnp.float32(0.0)
        for k in range(_NLEV):
            m = jnp.sum(sse_ref[k, :]) / n
            total = total + (_BETA * m + m)
        loss_ref[...] = jnp.full((8, 128), total, jnp.float32)
        probs = counts_ref[...] / jnp.float32(_NROWS)
        ent = jnp.sum(probs * jnp.log(probs + 1e-10), axis=1)
        perp = jnp.exp(-ent)
        perp_ref[...] = jnp.broadcast_to(perp[:, None], (8, 128))


def kernel(x, e0, e1, e2, e3):
    zf = x.reshape(_NROWS, _D)
    f32 = jnp.float32
    out_shapes = (
        jax.ShapeDtypeStruct((_NROWS, _D), f32),
        jax.ShapeDtypeStruct((_NROWS, _D), f32),
        jax.ShapeDtypeStruct((_NROWS, _D), f32),
        jax.ShapeDtypeStruct((_NROWS, _D), f32),
        jax.ShapeDtypeStruct((_NROWS, _D), f32),
        jax.ShapeDtypeStruct((_NBLK, 1, _BLK), jnp.int32),
        jax.ShapeDtypeStruct((_NBLK, 1, _BLK), jnp.int32),
        jax.ShapeDtypeStruct((_NBLK, 1, _BLK), jnp.int32),
        jax.ShapeDtypeStruct((_NBLK, 1, _BLK), jnp.int32),
        jax.ShapeDtypeStruct((8, 128), f32),
        jax.ShapeDtypeStruct((8, 128), f32),
    )
    row_spec = pl.BlockSpec((_BLK, _D), lambda i: (i, 0))
    emb_spec = pl.BlockSpec((_K, _D), lambda i: (0, 0))
    idx_spec = pl.BlockSpec((1, 1, _BLK), lambda i: (i, 0, 0))
    scl_spec = pl.BlockSpec((8, 128), lambda i: (0, 0))
    outs = pl.pallas_call(
        _rvq_kernel,
        grid=(_NBLK,),
        in_specs=[row_spec, emb_spec, emb_spec, emb_spec, emb_spec],
        out_specs=(row_spec, row_spec, row_spec, row_spec, row_spec,
                   idx_spec, idx_spec, idx_spec, idx_spec,
                   scl_spec, scl_spec),
        out_shape=out_shapes,
        scratch_shapes=[pltpu.VMEM((8, _K), f32),
                        pltpu.VMEM((8, _D), f32),
                        pltpu.VMEM((_NLEV, _K, 3 * _D), jnp.bfloat16),
                        pltpu.VMEM((8, _K), f32)],
        compiler_params=pltpu.CompilerParams(
            dimension_semantics=("arbitrary",)),
    )(zf, e0, e1, e2, e3)
    fq, q0, q1, q2, q3, i0, i1, i2, i3, loss, perp = outs
    shp = x.shape
    ishp = x.shape[:-1]
    qs = tuple(q.reshape(shp) for q in (q0, q1, q2, q3))
    idxs = tuple(ii.reshape(ishp) for ii in (i0, i1, i2, i3))
    perps = tuple(perp[k, 0] for k in range(_NLEV))
    return (fq.reshape(shp), loss[0, 0], (perps, qs, idxs))
